# baseline (speedup 1.0000x reference)
import jax
import jax.numpy as jnp
from jax import lax
from jax.experimental import pallas as pl
from jax.experimental.pallas import tpu as pltpu

_CH = 400
_NBUF = 3


def _body(lap_ref, fbf_ref, w1t_ref, w2t_ref, b_ref, out_ref,
          fk_ref, *scr):
    bufs = scr[:_NBUF]
    obufs = scr[_NBUF:2 * _NBUF]
    isems = scr[2 * _NBUF]
    osems = scr[2 * _NBUF + 1]
    fsem = scr[2 * _NBUF + 2]
    n = lap_ref.shape[0]
    hc = _CH // 2
    nchunk = n // _CH
    ngroups = (nchunk - 1) // _NBUF

    def fill_half(chunk, j, h):
        return pltpu.make_async_copy(
            lap_ref.at[pl.ds(chunk * _CH + h * hc, hc)],
            bufs[j].at[pl.ds(h * hc, hc)],
            isems.at[j, h])

    def fill_start(chunk, j):
        fill_half(chunk, j, 0).start()
        fill_half(chunk, j, 1).start()

    def fill_wait(chunk, j):
        fill_half(chunk, j, 0).wait()
        fill_half(chunk, j, 1).wait()

    def flush(chunk, j):
        return pltpu.make_async_copy(
            obufs[j], out_ref.at[pl.ds(chunk * _CH, _CH)], osems.at[j])

    for j in range(_NBUF):
        fill_start(j, j)
    pltpu.make_async_copy(fbf_ref, fk_ref, fsem).start()
    pltpu.make_async_copy(fbf_ref, fk_ref, fsem).wait()
    fk = fk_ref[...]
    w1t = w1t_ref[...]
    w2t = w2t_ref[...]
    b = b_ref[...]

    def chunk_compute(i, j):
        x = jnp.dot(bufs[j][...].astype(jnp.bfloat16), fk,
                    preferred_element_type=jnp.float32)
        f = fk_ref[pl.ds(i * _CH, _CH), :]
        return (
            jnp.dot((f + x).astype(jnp.bfloat16), w1t,
                    preferred_element_type=jnp.float32)
            + jnp.dot((x * f).astype(jnp.bfloat16), w2t,
                      preferred_element_type=jnp.float32)
            + b
        )

    def group(g, carry):
        for j in range(_NBUF):
            i = g * _NBUF + j
            fill_wait(i, j)
            pj = (j - 1) % _NBUF
            nxt = i + _NBUF - 1
            @pl.when((i >= 1) & (nxt < nchunk))
            def _():
                fill_start(nxt, pj)
            res = chunk_compute(i, j)
            @pl.when(i >= _NBUF)
            def _():
                flush(0, j).wait()
            obufs[j][...] = res
            flush(i, j).start()
        return carry

    lax.fori_loop(0, ngroups, group, 0)

    pc = nchunk - 1
    j = pc % _NBUF
    fill_wait(pc, j)
    res = chunk_compute(pc, j)
    flush(0, j).wait()
    obufs[j][...] = res
    flush(pc, j).start()
    for j in range(_NBUF):
        flush(0, j).wait()


def kernel(lap_matrix, eye_matrix, features, W1, b1, W2, b2):
    del eye_matrix
    n, d = features.shape

    feat_bf = features.astype(jnp.bfloat16)
    w1t = W1.T.astype(jnp.bfloat16)
    w2t = W2.T.astype(jnp.bfloat16)
    bias = (b1 + b2).reshape(1, d)

    return pl.pallas_call(
        _body,
        in_specs=[
            pl.BlockSpec(memory_space=pl.ANY),
            pl.BlockSpec(memory_space=pl.ANY),
            pl.BlockSpec(memory_space=pltpu.VMEM),
            pl.BlockSpec(memory_space=pltpu.VMEM),
            pl.BlockSpec(memory_space=pltpu.VMEM),
        ],
        out_specs=pl.BlockSpec(memory_space=pl.ANY),
        out_shape=jax.ShapeDtypeStruct((n, d), jnp.float32),
        scratch_shapes=(
            [pltpu.VMEM((n, d), jnp.bfloat16)]
            + [pltpu.VMEM((_CH, n), jnp.float32) for _ in range(_NBUF)]
            + [pltpu.VMEM((_CH, d), jnp.float32) for _ in range(_NBUF)]
            + [pltpu.SemaphoreType.DMA((_NBUF, 2)),
               pltpu.SemaphoreType.DMA((_NBUF,)),
               pltpu.SemaphoreType.DMA]
        ),
    )(lap_matrix, feat_bf, w1t, w2t, bias)


# speedup vs baseline: 1.0528x; 1.0528x over previous
import jax
import jax.numpy as jnp
from jax.experimental import pallas as pl
from jax.experimental.pallas import tpu as pltpu


_NQ = 2
_HM = 200


def _body(*refs):
    l_refs = refs[:_NQ]
    fk_ref, w1t_ref, w2t_ref, b_ref, out_ref = refs[_NQ:]
    hm = l_refs[0].shape[0]
    band = fk_ref.shape[0] // _NQ
    fk = fk_ref[...]
    w1t = w1t_ref[...]
    w2t = w2t_ref[...]
    b = b_ref[...]
    m = pl.program_id(0)
    for i, l_ref in enumerate(l_refs):
        x = jnp.dot(l_ref[...].astype(jnp.bfloat16), fk,
                    preferred_element_type=jnp.float32)
        f = fk_ref[pl.ds(i * band + m * hm, hm), :]
        out_ref[i, :, :] = (
            jnp.dot((f + x).astype(jnp.bfloat16), w1t,
                    preferred_element_type=jnp.float32)
            + jnp.dot((x * f).astype(jnp.bfloat16), w2t,
                      preferred_element_type=jnp.float32)
            + b
        )


def kernel(lap_matrix, eye_matrix, features, W1, b1, W2, b2):
    del eye_matrix
    n, d = features.shape
    g = n // (_NQ * _HM)

    feat_bf = features.astype(jnp.bfloat16)
    w1t = W1.T.astype(jnp.bfloat16)
    w2t = W2.T.astype(jnp.bfloat16)
    bias = (b1 + b2).reshape(1, d)

    out = pl.pallas_call(
        _body,
        grid=(g,),
        in_specs=[
            pl.BlockSpec((_HM, n), lambda m, j=j: (j * g + m, 0))
            for j in range(_NQ)
        ] + [
            pl.BlockSpec((n, d), lambda m: (0, 0)),
            pl.BlockSpec((d, d), lambda m: (0, 0)),
            pl.BlockSpec((d, d), lambda m: (0, 0)),
            pl.BlockSpec((1, d), lambda m: (0, 0)),
        ],
        out_specs=pl.BlockSpec((_NQ, _HM, d), lambda m: (0, m, 0)),
        out_shape=jax.ShapeDtypeStruct((_NQ, n // _NQ, d), jnp.float32),
        compiler_params=pltpu.CompilerParams(
            dimension_semantics=("arbitrary",),
        ),
    )(*([lap_matrix] * _NQ), feat_bf, w1t, w2t, bias)
    return out.reshape(n, d)


# speedup vs baseline: 1.1115x; 1.0557x over previous
import jax
import jax.numpy as jnp
from jax import lax
from jax.experimental import pallas as pl
from jax.experimental.pallas import tpu as pltpu


_NQ = 2
_HM = 200

_DN_T = (((1,), (1,)), ((), ()))


def _body(*refs):
    l_refs = refs[:_NQ]
    f_ref, w1_ref, w2_ref, b1_ref, b2_ref, out_ref, fkb_ref = refs[_NQ:]
    hm = l_refs[0].shape[0]
    band = f_ref.shape[0] // _NQ
    m = pl.program_id(0)

    @pl.when(m == 0)
    def _():
        fkb_ref[...] = f_ref[...].astype(jnp.bfloat16)

    fkb = fkb_ref[...]
    w1 = w1_ref[...].astype(jnp.bfloat16)
    w2 = w2_ref[...].astype(jnp.bfloat16)
    b = (b1_ref[...] + b2_ref[...]).reshape(1, -1)
    for i, l_ref in enumerate(l_refs):
        x = jnp.dot(l_ref[...].astype(jnp.bfloat16), fkb,
                    preferred_element_type=jnp.float32)
        f = f_ref[pl.ds(i * band + m * hm, hm), :]
        out_ref[i, :, :] = (
            lax.dot_general((f + x).astype(jnp.bfloat16), w1, _DN_T,
                            preferred_element_type=jnp.float32)
            + lax.dot_general((x * f).astype(jnp.bfloat16), w2, _DN_T,
                              preferred_element_type=jnp.float32)
            + b
        )


def kernel(lap_matrix, eye_matrix, features, W1, b1, W2, b2):
    del eye_matrix
    n, d = features.shape
    g = n // (_NQ * _HM)

    out = pl.pallas_call(
        _body,
        grid=(g,),
        in_specs=[
            pl.BlockSpec((_HM, n), lambda m, j=j: (j * g + m, 0))
            for j in range(_NQ)
        ] + [
            pl.BlockSpec((n, d), lambda m: (0, 0)),
            pl.BlockSpec((d, d), lambda m: (0, 0)),
            pl.BlockSpec((d, d), lambda m: (0, 0)),
            pl.BlockSpec((d,), lambda m: (0,)),
            pl.BlockSpec((d,), lambda m: (0,)),
        ],
        out_specs=pl.BlockSpec((_NQ, _HM, d), lambda m: (0, m, 0)),
        out_shape=jax.ShapeDtypeStruct((_NQ, n // _NQ, d), jnp.float32),
        scratch_shapes=[pltpu.VMEM((n, d), jnp.bfloat16)],
        compiler_params=pltpu.CompilerParams(
            dimension_semantics=("arbitrary",),
        ),
    )(*([lap_matrix] * _NQ), features, W1, W2, b1, b2)
    return out.reshape(n, d)


# speedup vs baseline: 1.1153x; 1.0034x over previous
import jax
import jax.numpy as jnp
from jax import lax
from jax.experimental import pallas as pl
from jax.experimental.pallas import tpu as pltpu


_NQ = 2
_HM = 200

_DN_T = (((1,), (1,)), ((), ()))


def _body(*refs):
    l_refs = refs[:_NQ]
    f_ref, w1_ref, w2_ref, b1_ref, b2_ref, out_ref, fkb_ref = refs[_NQ:]
    hm = l_refs[0].shape[0]
    band = f_ref.shape[0] // _NQ
    m = pl.program_id(0)

    @pl.when(m == 0)
    def _():
        fkb_ref[...] = f_ref[...].astype(jnp.bfloat16)

    fkb = fkb_ref[...]
    w1 = w1_ref[...].astype(jnp.bfloat16)
    w2 = w2_ref[...].astype(jnp.bfloat16)
    b = (b1_ref[...] + b2_ref[...]).reshape(1, -1)
    for i, l_ref in enumerate(l_refs):
        x = jnp.dot(l_ref[...].astype(jnp.bfloat16), fkb,
                    preferred_element_type=jnp.float32)
        f = f_ref[pl.ds(m * _NQ * hm + i * hm, hm), :]
        out_ref[pl.ds(i * hm, hm), :] = (
            lax.dot_general((f + x).astype(jnp.bfloat16), w1, _DN_T,
                            preferred_element_type=jnp.float32)
            + lax.dot_general((x * f).astype(jnp.bfloat16), w2, _DN_T,
                              preferred_element_type=jnp.float32)
            + b
        )


def kernel(lap_matrix, eye_matrix, features, W1, b1, W2, b2):
    del eye_matrix
    n, d = features.shape
    g = n // (_NQ * _HM)

    out = pl.pallas_call(
        _body,
        grid=(g,),
        in_specs=[
            pl.BlockSpec((_HM, n), lambda m, j=j: (_NQ * m + j, 0))
            for j in range(_NQ)
        ] + [
            pl.BlockSpec((n, d), lambda m: (0, 0)),
            pl.BlockSpec((d, d), lambda m: (0, 0)),
            pl.BlockSpec((d, d), lambda m: (0, 0)),
            pl.BlockSpec((d,), lambda m: (0,)),
            pl.BlockSpec((d,), lambda m: (0,)),
        ],
        out_specs=pl.BlockSpec((_NQ * _HM, d), lambda m: (m, 0)),
        out_shape=jax.ShapeDtypeStruct((n, d), jnp.float32),
        scratch_shapes=[pltpu.VMEM((n, d), jnp.bfloat16)],
        compiler_params=pltpu.CompilerParams(
            dimension_semantics=("arbitrary",),
        ),
    )(*([lap_matrix] * _NQ), features, W1, W2, b1, b2)
    return out
